# baseline (speedup 1.0000x reference)
import functools

import jax
import jax.numpy as jnp
from jax import lax
from jax.experimental import pallas as pl
from jax.experimental.pallas import tpu as pltpu
from jax.experimental.pallas import tpu_sc as plsc

F32 = jnp.float32
I32 = jnp.int32

N = 10000
D = 128
NP = 10240
RPT = NP // 16
NC = 2
NS = 16
B = 64
BD = 128
RB = 1000


def _deg_kernel(nb):
    mesh = plsc.VectorSubcoreMesh(core_axis_name="c", subcore_axis_name="s")

    @functools.partial(
        pl.kernel,
        out_type=jax.ShapeDtypeStruct((NC, NP), F32),
        mesh=mesh,
        scratch_types=[
            pltpu.VMEM_SHARED((NP,), F32),
            pltpu.VMEM((BD,), I32),
            pltpu.VMEM((BD,), F32),
            pltpu.VMEM((RPT,), F32),
        ],
    )
    def deg(row_h, w_h, out_h, acc, row_v, w_v, zbuf):
        cid = lax.axis_index("c")
        sid = lax.axis_index("s")
        wid = sid * NC + cid
        r0 = sid * RPT
        zf = jnp.zeros((16,), F32)
        for i in range(RPT // 16):
            zbuf[pl.ds(i * 16, 16)] = zf
        pltpu.sync_copy(zbuf, acc.at[pl.ds(r0, RPT)])
        plsc.subcore_barrier()

        def batch(b, c):
            base = wid * (nb * BD) + b * BD
            pltpu.sync_copy(row_h.at[pl.ds(base, BD)], row_v)
            pltpu.sync_copy(w_h.at[pl.ds(base, BD)], w_v)
            pltpu.sync_copy(w_v, acc.at[row_v], add=True)
            return c

        lax.fori_loop(0, nb, batch, 0)
        plsc.subcore_barrier()
        pltpu.sync_copy(acc.at[pl.ds(r0, RPT)], out_h.at[cid, pl.ds(r0, RPT)])

    return deg


def _a_w_kernel(nb):
    mesh = plsc.VectorSubcoreMesh(core_axis_name="c", subcore_axis_name="s")

    @functools.partial(
        pl.kernel,
        out_type=jax.ShapeDtypeStruct((NC, NP, D), F32),
        mesh=mesh,
        scratch_types=[
            pltpu.VMEM_SHARED((NP, D), F32),
            pltpu.VMEM((2, B), I32),
            pltpu.VMEM((2, B), I32),
            pltpu.VMEM((2, B), I32),
            pltpu.VMEM((2, B), I32),
            pltpu.VMEM((B,), F32),
            pltpu.VMEM((B,), F32),
            pltpu.VMEM((B,), F32),
            pltpu.VMEM((B,), F32),
            pltpu.VMEM((B, D), F32),
            pltpu.VMEM((B, D), F32),
            pltpu.VMEM((B, D), F32),
            pltpu.VMEM((B, D), F32),
            pltpu.SemaphoreType.DMA,
            pltpu.SemaphoreType.DMA,
            pltpu.SemaphoreType.DMA,
            pltpu.SemaphoreType.DMA,
            pltpu.SemaphoreType.DMA,
            pltpu.SemaphoreType.DMA,
            pltpu.SemaphoreType.DMA,
            pltpu.SemaphoreType.DMA,
            pltpu.SemaphoreType.DMA,
            pltpu.SemaphoreType.DMA,
            pltpu.SemaphoreType.DMA,
            pltpu.SemaphoreType.DMA,
        ],
    )
    def a_w(y_h, pk_h, wt_h, out_h, acc,
            pk0, pk1, pk2, pk3, wv0, wv1, wv2, wv3, g0, g1, g2, g3,
            si0, si1, si2, si3, sg0, sg1, sg2, sg3, ss0, ss1, ss2, ss3):
        cid = lax.axis_index("c")
        sid = lax.axis_index("s")
        wid = sid * NC + cid
        r0 = sid * RPT
        tb = wid * nb
        pks = [pk0, pk1, pk2, pk3]
        wvs = [wv0, wv1, wv2, wv3]
        gs = [g0, g1, g2, g3]
        semI = [si0, si1, si2, si3]
        semG = [sg0, sg1, sg2, sg3]
        semS = [ss0, ss1, ss2, ss3]
        zf = jnp.zeros((16,), F32)
        zi = jnp.zeros((16,), I32)

        for g in gs:
            for e in range(B):
                for j in range(D // 16):
                    g[e, pl.ds(j * 16, 16)] = zf
        for r in range(2):
            for c in range(B // 16):
                pk3[r, pl.ds(c * 16, 16)] = zi
        for i in range(RPT // B):
            pltpu.sync_copy(g0, acc.at[pl.ds(r0 + i * B, B)])
        rem = RPT - (RPT // B) * B
        if rem:
            pltpu.sync_copy(g0.at[pl.ds(0, rem)],
                            acc.at[pl.ds(r0 + (RPT // B) * B, rem)])
        plsc.subcore_barrier()

        pltpu.async_copy(g3, acc.at[pk3.at[1]], ss3, add=True)
        pltpu.sync_copy(pk_h.at[tb], pk0)
        pltpu.sync_copy(wt_h.at[tb], wv0)
        pltpu.async_copy(y_h.at[pk0.at[0]], g0, sg0)
        pltpu.async_copy(pk_h.at[tb + 1], pk1, si1)
        pltpu.async_copy(wt_h.at[tb + 1], wv1, si1)
        pltpu.async_copy(pk_h.at[tb + 2], pk2, si2)
        pltpu.async_copy(wt_h.at[tb + 2], wv2, si2)

        def phase(b, k):
            m, m1, m3 = k, (k + 1) % 4, (k + 3) % 4
            pltpu.make_async_copy(y_h.at[pks[m].at[0]], gs[m], semG[m]).wait()
            pltpu.make_async_copy(pk_h.at[tb], pks[m1], semI[m1]).wait()
            pltpu.make_async_copy(wt_h.at[tb], wvs[m1], semI[m1]).wait()
            pltpu.async_copy(y_h.at[pks[m1].at[0]], gs[m1], semG[m1])
            for ch in range(B // 16):
                wv = wvs[m][pl.ds(ch * 16, 16)]
                for l in range(16):
                    ws = jnp.full((16,), wv[l], F32)
                    e = ch * 16 + l
                    for j in range(D // 16):
                        sj = pl.ds(j * 16, 16)
                        gs[m][e, sj] = gs[m][e, sj] * ws
            pltpu.async_copy(gs[m], acc.at[pks[m].at[1]], semS[m], add=True)
            pltpu.make_async_copy(gs[m3], acc.at[pks[m3].at[1]], semS[m3]).wait()
            pltpu.async_copy(pk_h.at[tb + b + 3], pks[m3], semI[m3])
            pltpu.async_copy(wt_h.at[tb + b + 3], wvs[m3], semI[m3])

        def quad(gi, c):
            for k in range(4):
                phase(gi * 4 + k, k)
            return c

        lax.fori_loop(0, nb // 4, quad, 0)
        pltpu.make_async_copy(gs[3], acc.at[pk3.at[1]], ss3).wait()
        pltpu.make_async_copy(y_h.at[pk0.at[0]], g0, sg0).wait()
        pltpu.make_async_copy(pk_h.at[tb], pk1, si1).wait()
        pltpu.make_async_copy(wt_h.at[tb], wv1, si1).wait()
        pltpu.make_async_copy(pk_h.at[tb], pk2, si2).wait()
        pltpu.make_async_copy(wt_h.at[tb], wv2, si2).wait()
        plsc.subcore_barrier()
        pltpu.sync_copy(acc.at[pl.ds(r0, RPT)], out_h.at[cid, pl.ds(r0, RPT)])

    return a_w


def _prep_body(dg_ref, x_ref, dis_ref, y_ref):
    deg = dg_ref[0] + dg_ref[1]
    dis = jnp.where(deg > 0.0, lax.rsqrt(jnp.maximum(deg, 1e-12)), 0.0)
    dis_ref[...] = dis
    y_ref[...] = dis * x_ref[...]


def _mid_body(z_ref, dis_ref, o_ref):
    z = z_ref[...]
    dis = dis_ref[...]
    o_ref[...] = (dis * dis) * (z[0] + z[1])


def _fin_body(x_ref, dis_ref, z1_ref, z2_ref, w0_ref, w1_ref, w2_ref,
              b_ref, wc_ref, o_ref):
    dis_b = dis_ref[...]
    z1 = z1_ref[...]
    z2 = z2_ref[...]
    a = dis_b * (z1[0] + z1[1])
    bb = dis_b * (z2[0] + z2[1])
    x = x_ref[...]
    dot = functools.partial(jnp.dot, preferred_element_type=F32,
                            precision=lax.Precision.HIGHEST)
    pre = dot(x, w0_ref[...]) + dot(a, w1_ref[...]) + dot(bb, w2_ref[...])
    pre = pre + b_ref[...]
    gi = jax.nn.sigmoid(pre[:, :D])
    gt = jnp.tanh(pre[:, D:2 * D])
    cn = gi * gt
    go = jax.nn.sigmoid(pre[:, 2 * D:] + wc_ref[...] * cn)
    o_ref[...] = go * jnp.tanh(cn)


def kernel(X, edge_weight, Wx, bx, Wh, bh, w_c, b_gate, edge_index):
    E = edge_weight.shape[0]
    row0, col0 = edge_index[0], edge_index[1]

    nb_d = -(-E // (NC * NS * BD))
    e_pad_d = NC * NS * BD * nb_d
    rowf = jnp.concatenate([row0, jnp.zeros((e_pad_d - E,), I32)])
    wf = jnp.concatenate([edge_weight, jnp.zeros((e_pad_d - E,), F32)])

    nb = -(-E // (NC * NS * B))
    nb = -(-nb // 4) * 4
    nbt = NC * NS * nb + 3
    ep = nbt * B
    colp = jnp.concatenate([col0, jnp.zeros((ep - E,), I32)]).reshape(nbt, 1, B)
    rowp = jnp.concatenate([row0, jnp.zeros((ep - E,), I32)]).reshape(nbt, 1, B)
    wt = jnp.concatenate([edge_weight, jnp.zeros((ep - E,), F32)]).reshape(nbt, B)
    pk = jnp.concatenate([colp, rowp], axis=1)

    degp = _deg_kernel(nb_d)(rowf, wf)

    dis2d, y1 = pl.pallas_call(
        _prep_body,
        grid=(N // RB,),
        in_specs=[
            pl.BlockSpec((NC, RB, 1), lambda i: (0, i, 0)),
            pl.BlockSpec((RB, D), lambda i: (i, 0)),
        ],
        out_specs=[
            pl.BlockSpec((RB, 1), lambda i: (i, 0)),
            pl.BlockSpec((RB, D), lambda i: (i, 0)),
        ],
        out_shape=[
            jax.ShapeDtypeStruct((N, 1), F32),
            jax.ShapeDtypeStruct((N, D), F32),
        ],
    )(degp.reshape(NC, NP, 1), X)

    a_w = _a_w_kernel(nb)

    z1p = a_w(y1, pk, wt)

    t2 = pl.pallas_call(
        _mid_body,
        grid=(N // RB,),
        in_specs=[
            pl.BlockSpec((NC, RB, D), lambda i: (0, i, 0)),
            pl.BlockSpec((RB, 1), lambda i: (i, 0)),
        ],
        out_specs=pl.BlockSpec((RB, D), lambda i: (i, 0)),
        out_shape=jax.ShapeDtypeStruct((N, D), F32),
    )(z1p, dis2d)

    z2p = a_w(t2, pk, wt)

    g_idx = jnp.array([0, 2, 3])
    Wg = Wx[g_idx]
    w0 = jnp.transpose(Wg[:, 0] - Wg[:, 2], (1, 0, 2)).reshape(D, 3 * D)
    w1 = jnp.transpose(-Wg[:, 1], (1, 0, 2)).reshape(D, 3 * D)
    w2 = jnp.transpose(2.0 * Wg[:, 2], (1, 0, 2)).reshape(D, 3 * D)
    bcat = (bx[g_idx] + bh[g_idx] + b_gate[g_idx, 0]).reshape(1, 3 * D)
    wc2 = w_c[2].reshape(1, D)

    return pl.pallas_call(
        _fin_body,
        grid=(N // RB,),
        in_specs=[
            pl.BlockSpec((RB, D), lambda i: (i, 0)),
            pl.BlockSpec((RB, 1), lambda i: (i, 0)),
            pl.BlockSpec((NC, RB, D), lambda i: (0, i, 0)),
            pl.BlockSpec((NC, RB, D), lambda i: (0, i, 0)),
            pl.BlockSpec((D, 3 * D), lambda i: (0, 0)),
            pl.BlockSpec((D, 3 * D), lambda i: (0, 0)),
            pl.BlockSpec((D, 3 * D), lambda i: (0, 0)),
            pl.BlockSpec((1, 3 * D), lambda i: (0, 0)),
            pl.BlockSpec((1, D), lambda i: (0, 0)),
        ],
        out_specs=pl.BlockSpec((RB, D), lambda i: (i, 0)),
        out_shape=jax.ShapeDtypeStruct((N, D), F32),
    )(X, dis2d, z1p, z2p, w0, w1, w2, bcat, wc2)


# speedup vs baseline: 1.0218x; 1.0218x over previous
import functools

import jax
import jax.numpy as jnp
from jax import lax
from jax.experimental import pallas as pl
from jax.experimental.pallas import tpu as pltpu
from jax.experimental.pallas import tpu_sc as plsc

F32 = jnp.float32
I32 = jnp.int32

N = 10000
D = 128
NP = 10240
RPT = NP // 16
NC = 2
NS = 16
B = 64
BD = 128
RB = 1000


def _deg_kernel(nb):
    mesh = plsc.VectorSubcoreMesh(core_axis_name="c", subcore_axis_name="s")

    @functools.partial(
        pl.kernel,
        out_type=jax.ShapeDtypeStruct((NC, NP), F32),
        mesh=mesh,
        scratch_types=[
            pltpu.VMEM_SHARED((NP,), F32),
            pltpu.VMEM((BD,), I32),
            pltpu.VMEM((BD,), F32),
            pltpu.VMEM((RPT,), F32),
        ],
    )
    def deg(row_h, w_h, out_h, acc, row_v, w_v, zbuf):
        cid = lax.axis_index("c")
        sid = lax.axis_index("s")
        wid = sid * NC + cid
        r0 = sid * RPT
        zf = jnp.zeros((16,), F32)
        for i in range(RPT // 16):
            zbuf[pl.ds(i * 16, 16)] = zf
        pltpu.sync_copy(zbuf, acc.at[pl.ds(r0, RPT)])
        plsc.subcore_barrier()

        def batch(b, c):
            base = wid * (nb * BD) + b * BD
            pltpu.sync_copy(row_h.at[pl.ds(base, BD)], row_v)
            pltpu.sync_copy(w_h.at[pl.ds(base, BD)], w_v)
            pltpu.sync_copy(w_v, acc.at[row_v], add=True)
            return c

        lax.fori_loop(0, nb, batch, 0)
        plsc.subcore_barrier()
        pltpu.sync_copy(acc.at[pl.ds(r0, RPT)], out_h.at[cid, pl.ds(r0, RPT)])

    return deg


def _a_w_kernel(nb):
    mesh = plsc.VectorSubcoreMesh(core_axis_name="c", subcore_axis_name="s")

    @functools.partial(
        pl.kernel,
        out_type=jax.ShapeDtypeStruct((NC, NP, D), F32),
        mesh=mesh,
        scratch_types=[
            pltpu.VMEM_SHARED((NP, D), F32),
            pltpu.VMEM((2, B), I32),
            pltpu.VMEM((2, B), I32),
            pltpu.VMEM((2, B), I32),
            pltpu.VMEM((2, B), I32),
            pltpu.VMEM((B,), F32),
            pltpu.VMEM((B,), F32),
            pltpu.VMEM((B,), F32),
            pltpu.VMEM((B,), F32),
            pltpu.VMEM((B + 16,), F32),
            pltpu.VMEM((B, D), F32),
            pltpu.VMEM((B, D), F32),
            pltpu.VMEM((B, D), F32),
            pltpu.VMEM((B, D), F32),
            pltpu.SemaphoreType.DMA,
            pltpu.SemaphoreType.DMA,
            pltpu.SemaphoreType.DMA,
            pltpu.SemaphoreType.DMA,
            pltpu.SemaphoreType.DMA,
            pltpu.SemaphoreType.DMA,
            pltpu.SemaphoreType.DMA,
            pltpu.SemaphoreType.DMA,
            pltpu.SemaphoreType.DMA,
            pltpu.SemaphoreType.DMA,
            pltpu.SemaphoreType.DMA,
            pltpu.SemaphoreType.DMA,
        ],
    )
    def a_w(y_h, pk_h, wt_h, out_h, acc,
            pk0, pk1, pk2, pk3, wv0, wv1, wv2, wv3, wvp, g0, g1, g2, g3,
            si0, si1, si2, si3, sg0, sg1, sg2, sg3, ss0, ss1, ss2, ss3):
        cid = lax.axis_index("c")
        sid = lax.axis_index("s")
        wid = sid * NC + cid
        r0 = sid * RPT
        tb = wid * nb
        pks = [pk0, pk1, pk2, pk3]
        wvs = [wv0, wv1, wv2, wv3]
        gs = [g0, g1, g2, g3]
        semI = [si0, si1, si2, si3]
        semG = [sg0, sg1, sg2, sg3]
        semS = [ss0, ss1, ss2, ss3]
        zf = jnp.zeros((16,), F32)
        zi = jnp.zeros((16,), I32)

        def zrow(e, c):
            for g in gs:
                for j in range(D // 16):
                    g[e, pl.ds(j * 16, 16)] = zf
            return c

        lax.fori_loop(0, B, zrow, 0)
        for r in range(2):
            for c in range(B // 16):
                pk3[r, pl.ds(c * 16, 16)] = zi
        for i in range(RPT // B):
            pltpu.sync_copy(g0, acc.at[pl.ds(r0 + i * B, B)])
        rem = RPT - (RPT // B) * B
        if rem:
            pltpu.sync_copy(g0.at[pl.ds(0, rem)],
                            acc.at[pl.ds(r0 + (RPT // B) * B, rem)])
        plsc.subcore_barrier()

        pltpu.async_copy(g3, acc.at[pk3.at[1]], ss3, add=True)
        pltpu.sync_copy(pk_h.at[tb], pk0)
        pltpu.sync_copy(wt_h.at[tb], wv0)
        pltpu.async_copy(y_h.at[pk0.at[0]], g0, sg0)
        pltpu.async_copy(pk_h.at[tb + 1], pk1, si1)
        pltpu.async_copy(wt_h.at[tb + 1], wv1, si1)
        pltpu.async_copy(pk_h.at[tb + 2], pk2, si2)
        pltpu.async_copy(wt_h.at[tb + 2], wv2, si2)

        def phase(b, k):
            m, m1, m3 = k, (k + 1) % 4, (k + 3) % 4
            pltpu.make_async_copy(y_h.at[pks[m].at[0]], gs[m], semG[m]).wait()
            pltpu.make_async_copy(pk_h.at[tb], pks[m1], semI[m1]).wait()
            pltpu.make_async_copy(wt_h.at[tb], wvs[m1], semI[m1]).wait()
            pltpu.async_copy(y_h.at[pks[m1].at[0]], gs[m1], semG[m1])
            for c4 in range(B // 16):
                s4 = pl.ds(c4 * 16, 16)
                wvp[s4] = wvs[m][s4]

            def srow(i, c):
                for u in range(2):
                    e = i * 2 + u
                    ws = jnp.full((16,), wvp[pl.ds(e, 16)][0], F32)
                    for j in range(D // 16):
                        sj = pl.ds(j * 16, 16)
                        gs[m][e, sj] = gs[m][e, sj] * ws
                return c

            lax.fori_loop(0, B // 2, srow, 0)
            pltpu.async_copy(gs[m], acc.at[pks[m].at[1]], semS[m], add=True)
            pltpu.make_async_copy(gs[m3], acc.at[pks[m3].at[1]], semS[m3]).wait()
            pltpu.async_copy(pk_h.at[tb + b + 3], pks[m3], semI[m3])
            pltpu.async_copy(wt_h.at[tb + b + 3], wvs[m3], semI[m3])

        def quad(gi, c):
            for k in range(4):
                phase(gi * 4 + k, k)
            return c

        lax.fori_loop(0, nb // 4, quad, 0)
        pltpu.make_async_copy(gs[3], acc.at[pk3.at[1]], ss3).wait()
        pltpu.make_async_copy(y_h.at[pk0.at[0]], g0, sg0).wait()
        pltpu.make_async_copy(pk_h.at[tb], pk1, si1).wait()
        pltpu.make_async_copy(wt_h.at[tb], wv1, si1).wait()
        pltpu.make_async_copy(pk_h.at[tb], pk2, si2).wait()
        pltpu.make_async_copy(wt_h.at[tb], wv2, si2).wait()
        plsc.subcore_barrier()
        pltpu.sync_copy(acc.at[pl.ds(r0, RPT)], out_h.at[cid, pl.ds(r0, RPT)])

    return a_w


def _prep_body(dg_ref, x_ref, dis_ref, y_ref):
    deg = dg_ref[0] + dg_ref[1]
    dis = jnp.where(deg > 0.0, lax.rsqrt(jnp.maximum(deg, 1e-12)), 0.0)
    dis_ref[...] = dis
    y_ref[...] = dis * x_ref[...]


def _mid_body(z_ref, dis_ref, o_ref):
    z = z_ref[...]
    dis = dis_ref[...]
    o_ref[...] = (dis * dis) * (z[0] + z[1])


def _fin_body(x_ref, dis_ref, z1_ref, z2_ref, w0_ref, w1_ref, w2_ref,
              b_ref, wc_ref, o_ref):
    dis_b = dis_ref[...]
    z1 = z1_ref[...]
    z2 = z2_ref[...]
    a = dis_b * (z1[0] + z1[1])
    bb = dis_b * (z2[0] + z2[1])
    x = x_ref[...]
    dot = functools.partial(jnp.dot, preferred_element_type=F32,
                            precision=lax.Precision.HIGHEST)
    pre = dot(x, w0_ref[...]) + dot(a, w1_ref[...]) + dot(bb, w2_ref[...])
    pre = pre + b_ref[...]
    gi = jax.nn.sigmoid(pre[:, :D])
    gt = jnp.tanh(pre[:, D:2 * D])
    cn = gi * gt
    go = jax.nn.sigmoid(pre[:, 2 * D:] + wc_ref[...] * cn)
    o_ref[...] = go * jnp.tanh(cn)


def kernel(X, edge_weight, Wx, bx, Wh, bh, w_c, b_gate, edge_index):
    E = edge_weight.shape[0]
    row0, col0 = edge_index[0], edge_index[1]

    nb_d = -(-E // (NC * NS * BD))
    e_pad_d = NC * NS * BD * nb_d
    rowf = jnp.concatenate([row0, jnp.zeros((e_pad_d - E,), I32)])
    wf = jnp.concatenate([edge_weight, jnp.zeros((e_pad_d - E,), F32)])

    nb = -(-E // (NC * NS * B))
    nb = -(-nb // 4) * 4
    nbt = NC * NS * nb + 3
    ep = nbt * B
    colp = jnp.concatenate([col0, jnp.zeros((ep - E,), I32)]).reshape(nbt, 1, B)
    rowp = jnp.concatenate([row0, jnp.zeros((ep - E,), I32)]).reshape(nbt, 1, B)
    wt = jnp.concatenate([edge_weight, jnp.zeros((ep - E,), F32)]).reshape(nbt, B)
    pk = jnp.concatenate([colp, rowp], axis=1)

    degp = _deg_kernel(nb_d)(rowf, wf)

    dis2d, y1 = pl.pallas_call(
        _prep_body,
        grid=(N // RB,),
        in_specs=[
            pl.BlockSpec((NC, RB, 1), lambda i: (0, i, 0)),
            pl.BlockSpec((RB, D), lambda i: (i, 0)),
        ],
        out_specs=[
            pl.BlockSpec((RB, 1), lambda i: (i, 0)),
            pl.BlockSpec((RB, D), lambda i: (i, 0)),
        ],
        out_shape=[
            jax.ShapeDtypeStruct((N, 1), F32),
            jax.ShapeDtypeStruct((N, D), F32),
        ],
    )(degp.reshape(NC, NP, 1), X)

    a_w = _a_w_kernel(nb)

    z1p = a_w(y1, pk, wt)

    t2 = pl.pallas_call(
        _mid_body,
        grid=(N // RB,),
        in_specs=[
            pl.BlockSpec((NC, RB, D), lambda i: (0, i, 0)),
            pl.BlockSpec((RB, 1), lambda i: (i, 0)),
        ],
        out_specs=pl.BlockSpec((RB, D), lambda i: (i, 0)),
        out_shape=jax.ShapeDtypeStruct((N, D), F32),
    )(z1p, dis2d)

    z2p = a_w(t2, pk, wt)

    g_idx = jnp.array([0, 2, 3])
    Wg = Wx[g_idx]
    w0 = jnp.transpose(Wg[:, 0] - Wg[:, 2], (1, 0, 2)).reshape(D, 3 * D)
    w1 = jnp.transpose(-Wg[:, 1], (1, 0, 2)).reshape(D, 3 * D)
    w2 = jnp.transpose(2.0 * Wg[:, 2], (1, 0, 2)).reshape(D, 3 * D)
    bcat = (bx[g_idx] + bh[g_idx] + b_gate[g_idx, 0]).reshape(1, 3 * D)
    wc2 = w_c[2].reshape(1, D)

    return pl.pallas_call(
        _fin_body,
        grid=(N // RB,),
        in_specs=[
            pl.BlockSpec((RB, D), lambda i: (i, 0)),
            pl.BlockSpec((RB, 1), lambda i: (i, 0)),
            pl.BlockSpec((NC, RB, D), lambda i: (0, i, 0)),
            pl.BlockSpec((NC, RB, D), lambda i: (0, i, 0)),
            pl.BlockSpec((D, 3 * D), lambda i: (0, 0)),
            pl.BlockSpec((D, 3 * D), lambda i: (0, 0)),
            pl.BlockSpec((D, 3 * D), lambda i: (0, 0)),
            pl.BlockSpec((1, 3 * D), lambda i: (0, 0)),
            pl.BlockSpec((1, D), lambda i: (0, 0)),
        ],
        out_specs=pl.BlockSpec((RB, D), lambda i: (i, 0)),
        out_shape=jax.ShapeDtypeStruct((N, D), F32),
    )(X, dis2d, z1p, z2p, w0, w1, w2, bcat, wc2)


# speedup vs baseline: 1.4333x; 1.4027x over previous
import functools

import jax
import jax.numpy as jnp
from jax import lax
from jax.experimental import pallas as pl
from jax.experimental.pallas import tpu as pltpu
from jax.experimental.pallas import tpu_sc as plsc

F32 = jnp.float32
I32 = jnp.int32

N = 10000
D = 128
NP = 10240
RPT = NP // 16
NC = 2
NS = 16
B = 64
BD = 128
RB = 1000


def _deg_kernel(nb):
    mesh = plsc.VectorSubcoreMesh(core_axis_name="c", subcore_axis_name="s")

    @functools.partial(
        pl.kernel,
        out_type=jax.ShapeDtypeStruct((NC, NP), F32),
        mesh=mesh,
        scratch_types=[
            pltpu.VMEM_SHARED((NP,), F32),
            pltpu.VMEM((BD,), I32),
            pltpu.VMEM((BD,), F32),
            pltpu.VMEM((RPT,), F32),
        ],
    )
    def deg(row_h, w_h, out_h, acc, row_v, w_v, zbuf):
        cid = lax.axis_index("c")
        sid = lax.axis_index("s")
        wid = sid * NC + cid
        r0 = sid * RPT
        zf = jnp.zeros((16,), F32)
        for i in range(RPT // 16):
            zbuf[pl.ds(i * 16, 16)] = zf
        pltpu.sync_copy(zbuf, acc.at[pl.ds(r0, RPT)])
        plsc.subcore_barrier()

        def batch(b, c):
            base = wid * (nb * BD) + b * BD
            pltpu.sync_copy(row_h.at[pl.ds(base, BD)], row_v)
            pltpu.sync_copy(w_h.at[pl.ds(base, BD)], w_v)
            pltpu.sync_copy(w_v, acc.at[row_v], add=True)
            return c

        lax.fori_loop(0, nb, batch, 0)
        plsc.subcore_barrier()
        pltpu.sync_copy(acc.at[pl.ds(r0, RPT)], out_h.at[cid, pl.ds(r0, RPT)])

    return deg


def _a_w_kernel(nb0, nb1):
    mesh = plsc.VectorSubcoreMesh(core_axis_name="c", subcore_axis_name="s")

    @functools.partial(
        pl.kernel,
        out_type=jax.ShapeDtypeStruct((NC, NP, D), F32),
        mesh=mesh,
        scratch_types=[
            pltpu.VMEM_SHARED((NP, D), F32),
            pltpu.VMEM((2, B), I32),
            pltpu.VMEM((2, B), I32),
            pltpu.VMEM((2, B), I32),
            pltpu.VMEM((B,), F32),
            pltpu.VMEM((B,), F32),
            pltpu.VMEM((B,), F32),
            pltpu.VMEM((B, D), F32),
            pltpu.VMEM((B, D), F32),
            pltpu.VMEM((B, D), F32),
            pltpu.SemaphoreType.DMA,
            pltpu.SemaphoreType.DMA,
            pltpu.SemaphoreType.DMA,
            pltpu.SemaphoreType.DMA,
            pltpu.SemaphoreType.DMA,
            pltpu.SemaphoreType.DMA,
            pltpu.SemaphoreType.DMA,
            pltpu.SemaphoreType.DMA,
            pltpu.SemaphoreType.DMA,
        ],
    )
    def a_w(y_h, pk_h, wt_h, out_h, acc, pk0, pk1, pk2, wv0, wv1, wv2,
            g0, g1, g2, si0, si1, si2, sg0, sg1, sg2, ss0, ss1, ss2):
        cid = lax.axis_index("c")
        sid = lax.axis_index("s")
        r0 = sid * RPT
        nbc = jnp.where(cid == 0, nb0, nb1)
        tb = cid * (NS * nb0) + sid * nbc
        pks = [pk0, pk1, pk2]
        wvs = [wv0, wv1, wv2]
        gs = [g0, g1, g2]
        semI = [si0, si1, si2]
        semG = [sg0, sg1, sg2]
        semS = [ss0, ss1, ss2]
        zf = jnp.zeros((16,), F32)
        zi = jnp.zeros((16,), I32)

        for g in gs:
            for e in range(B):
                for j in range(D // 16):
                    g[e, pl.ds(j * 16, 16)] = zf
        for r in range(2):
            for c in range(B // 16):
                pk2[r, pl.ds(c * 16, 16)] = zi
        for i in range(RPT // B):
            pltpu.sync_copy(g0, acc.at[pl.ds(r0 + i * B, B)])
        plsc.subcore_barrier()

        pltpu.async_copy(g2, acc.at[pk2.at[1]], ss2, add=True)
        pltpu.sync_copy(pk_h.at[tb], pk0)
        pltpu.sync_copy(wt_h.at[tb], wv0)
        pltpu.async_copy(y_h.at[pk0.at[0]], g0, sg0)
        pltpu.async_copy(pk_h.at[tb + 1], pk1, si1)
        pltpu.async_copy(wt_h.at[tb + 1], wv1, si1)

        def phase(b, k):
            m, m1, m2 = k, (k + 1) % 3, (k + 2) % 3
            pltpu.make_async_copy(y_h.at[pks[m].at[0]], gs[m], semG[m]).wait()
            pltpu.make_async_copy(pk_h.at[tb], pks[m1], semI[m1]).wait()
            pltpu.make_async_copy(wt_h.at[tb], wvs[m1], semI[m1]).wait()
            pltpu.async_copy(y_h.at[pks[m1].at[0]], gs[m1], semG[m1])
            for ch in range(B // 16):
                wv = wvs[m][pl.ds(ch * 16, 16)]
                for l in range(16):
                    ws = jnp.full((16,), wv[l], F32)
                    e = ch * 16 + l
                    for j in range(D // 16):
                        sj = pl.ds(j * 16, 16)
                        gs[m][e, sj] = gs[m][e, sj] * ws
            pltpu.async_copy(gs[m], acc.at[pks[m].at[1]], semS[m], add=True)
            pltpu.make_async_copy(gs[m2], acc.at[pks[m2].at[1]], semS[m2]).wait()
            pltpu.async_copy(pk_h.at[tb + b + 2], pks[m2], semI[m2])
            pltpu.async_copy(wt_h.at[tb + b + 2], wvs[m2], semI[m2])

        def triple(gi, c):
            for k in range(3):
                phase(gi * 3 + k, k)
            return c

        lax.fori_loop(0, nbc // 3, triple, 0)
        pltpu.make_async_copy(gs[2], acc.at[pk2.at[1]], ss2).wait()
        pltpu.make_async_copy(y_h.at[pk0.at[0]], g0, sg0).wait()
        pltpu.make_async_copy(pk_h.at[tb], pk1, si1).wait()
        pltpu.make_async_copy(wt_h.at[tb], wv1, si1).wait()
        plsc.subcore_barrier()
        pltpu.sync_copy(acc.at[pl.ds(r0, RPT)], out_h.at[cid, pl.ds(r0, RPT)])

    return a_w


def _prep_body(dg_ref, x_ref, dis_ref, y_ref):
    deg = dg_ref[0] + dg_ref[1]
    dis = jnp.where(deg > 0.0, lax.rsqrt(jnp.maximum(deg, 1e-12)), 0.0)
    dis_ref[...] = dis
    y_ref[...] = dis * x_ref[...]


def _mid_body(z_ref, dis_ref, o_ref):
    z = z_ref[...]
    dis = dis_ref[...]
    o_ref[...] = (dis * dis) * (z[0] + z[1])


def _fin_body(x_ref, dis_ref, z1_ref, z2_ref, w0_ref, w1_ref, w2_ref,
              b_ref, wc_ref, o_ref):
    dis_b = dis_ref[...]
    z1 = z1_ref[...]
    z2 = z2_ref[...]
    a = dis_b * (z1[0] + z1[1])
    bb = dis_b * (z2[0] + z2[1])
    x = x_ref[...]
    dot = functools.partial(jnp.dot, preferred_element_type=F32,
                            precision=lax.Precision.HIGHEST)
    pre = dot(x, w0_ref[...]) + dot(a, w1_ref[...]) + dot(bb, w2_ref[...])
    pre = pre + b_ref[...]
    gi = jax.nn.sigmoid(pre[:, :D])
    gt = jnp.tanh(pre[:, D:2 * D])
    cn = gi * gt
    go = jax.nn.sigmoid(pre[:, 2 * D:] + wc_ref[...] * cn)
    o_ref[...] = go * jnp.tanh(cn)


def kernel(X, edge_weight, Wx, bx, Wh, bh, w_c, b_gate, edge_index):
    E = edge_weight.shape[0]
    row0, col0 = edge_index[0], edge_index[1]

    nb_d = -(-E // (NC * NS * BD))
    e_pad_d = NC * NS * BD * nb_d
    rowf = jnp.concatenate([row0, jnp.zeros((e_pad_d - E,), I32)])
    wf = jnp.concatenate([edge_weight, jnp.zeros((e_pad_d - E,), F32)])

    nb_pt = -(-E // (2 * NS * B))
    nb0 = -(-(2 * nb_pt * 35) // (100 * 3)) * 3
    nb1 = -(-(E - NS * nb0 * B) // (NS * B * 3)) * 3
    nbt = NS * (nb0 + nb1) + 2
    ep = nbt * B
    colp = jnp.concatenate([col0, jnp.zeros((ep - E,), I32)]).reshape(nbt, 1, B)
    rowp = jnp.concatenate([row0, jnp.zeros((ep - E,), I32)]).reshape(nbt, 1, B)
    wt = jnp.concatenate([edge_weight, jnp.zeros((ep - E,), F32)]).reshape(nbt, B)
    pk = jnp.concatenate([colp, rowp], axis=1)

    degp = _deg_kernel(nb_d)(rowf, wf)

    dis2d, y1 = pl.pallas_call(
        _prep_body,
        grid=(N // RB,),
        in_specs=[
            pl.BlockSpec((NC, RB, 1), lambda i: (0, i, 0)),
            pl.BlockSpec((RB, D), lambda i: (i, 0)),
        ],
        out_specs=[
            pl.BlockSpec((RB, 1), lambda i: (i, 0)),
            pl.BlockSpec((RB, D), lambda i: (i, 0)),
        ],
        out_shape=[
            jax.ShapeDtypeStruct((N, 1), F32),
            jax.ShapeDtypeStruct((N, D), F32),
        ],
    )(degp.reshape(NC, NP, 1), X)

    a_w = _a_w_kernel(nb0, nb1)

    z1p = a_w(y1, pk, wt)

    t2 = pl.pallas_call(
        _mid_body,
        grid=(N // RB,),
        in_specs=[
            pl.BlockSpec((NC, RB, D), lambda i: (0, i, 0)),
            pl.BlockSpec((RB, 1), lambda i: (i, 0)),
        ],
        out_specs=pl.BlockSpec((RB, D), lambda i: (i, 0)),
        out_shape=jax.ShapeDtypeStruct((N, D), F32),
    )(z1p, dis2d)

    z2p = a_w(t2, pk, wt)

    g_idx = jnp.array([0, 2, 3])
    Wg = Wx[g_idx]
    w0 = jnp.transpose(Wg[:, 0] - Wg[:, 2], (1, 0, 2)).reshape(D, 3 * D)
    w1 = jnp.transpose(-Wg[:, 1], (1, 0, 2)).reshape(D, 3 * D)
    w2 = jnp.transpose(2.0 * Wg[:, 2], (1, 0, 2)).reshape(D, 3 * D)
    bcat = (bx[g_idx] + bh[g_idx] + b_gate[g_idx, 0]).reshape(1, 3 * D)
    wc2 = w_c[2].reshape(1, D)

    return pl.pallas_call(
        _fin_body,
        grid=(N // RB,),
        in_specs=[
            pl.BlockSpec((RB, D), lambda i: (i, 0)),
            pl.BlockSpec((RB, 1), lambda i: (i, 0)),
            pl.BlockSpec((NC, RB, D), lambda i: (0, i, 0)),
            pl.BlockSpec((NC, RB, D), lambda i: (0, i, 0)),
            pl.BlockSpec((D, 3 * D), lambda i: (0, 0)),
            pl.BlockSpec((D, 3 * D), lambda i: (0, 0)),
            pl.BlockSpec((D, 3 * D), lambda i: (0, 0)),
            pl.BlockSpec((1, 3 * D), lambda i: (0, 0)),
            pl.BlockSpec((1, D), lambda i: (0, 0)),
        ],
        out_specs=pl.BlockSpec((RB, D), lambda i: (i, 0)),
        out_shape=jax.ShapeDtypeStruct((N, D), F32),
    )(X, dis2d, z1p, z2p, w0, w1, w2, bcat, wc2)


# speedup vs baseline: 1.5045x; 1.0497x over previous
import functools

import jax
import jax.numpy as jnp
from jax import lax
from jax.experimental import pallas as pl
from jax.experimental.pallas import tpu as pltpu
from jax.experimental.pallas import tpu_sc as plsc

F32 = jnp.float32
I32 = jnp.int32

N = 10000
D = 128
NP = 10240
RPT = NP // 16
NC = 2
NS = 16
B = 64
BD = 128
RB = 1000


def _deg_kernel(nb):
    mesh = plsc.VectorSubcoreMesh(core_axis_name="c", subcore_axis_name="s")

    @functools.partial(
        pl.kernel,
        out_type=jax.ShapeDtypeStruct((NC, NP), F32),
        mesh=mesh,
        scratch_types=[
            pltpu.VMEM_SHARED((NP,), F32),
            pltpu.VMEM((BD,), I32),
            pltpu.VMEM((BD,), F32),
            pltpu.VMEM((RPT,), F32),
        ],
    )
    def deg(row_h, w_h, out_h, acc, row_v, w_v, zbuf):
        cid = lax.axis_index("c")
        sid = lax.axis_index("s")
        wid = sid * NC + cid
        r0 = sid * RPT
        zf = jnp.zeros((16,), F32)
        for i in range(RPT // 16):
            zbuf[pl.ds(i * 16, 16)] = zf
        pltpu.sync_copy(zbuf, acc.at[pl.ds(r0, RPT)])
        plsc.subcore_barrier()

        def batch(b, c):
            base = wid * (nb * BD) + b * BD
            pltpu.sync_copy(row_h.at[pl.ds(base, BD)], row_v)
            pltpu.sync_copy(w_h.at[pl.ds(base, BD)], w_v)
            pltpu.sync_copy(w_v, acc.at[row_v], add=True)
            return c

        lax.fori_loop(0, nb, batch, 0)
        plsc.subcore_barrier()
        pltpu.sync_copy(acc.at[pl.ds(r0, RPT)], out_h.at[cid, pl.ds(r0, RPT)])

    return deg


def _a_w_kernel(nb0, nb1):
    mesh = plsc.VectorSubcoreMesh(core_axis_name="c", subcore_axis_name="s")

    @functools.partial(
        pl.kernel,
        out_type=jax.ShapeDtypeStruct((NC, NP, D), F32),
        mesh=mesh,
        scratch_types=[
            pltpu.VMEM_SHARED((NP, D), F32),
            pltpu.VMEM((2, B), I32),
            pltpu.VMEM((2, B), I32),
            pltpu.VMEM((2, B), I32),
            pltpu.VMEM((B,), F32),
            pltpu.VMEM((B,), F32),
            pltpu.VMEM((B,), F32),
            pltpu.VMEM((B, D), F32),
            pltpu.VMEM((B, D), F32),
            pltpu.VMEM((B, D), F32),
            pltpu.SemaphoreType.DMA,
            pltpu.SemaphoreType.DMA,
            pltpu.SemaphoreType.DMA,
            pltpu.SemaphoreType.DMA,
            pltpu.SemaphoreType.DMA,
            pltpu.SemaphoreType.DMA,
            pltpu.SemaphoreType.DMA,
            pltpu.SemaphoreType.DMA,
            pltpu.SemaphoreType.DMA,
        ],
    )
    def a_w(y_h, pk_h, wt_h, out_h, acc, pk0, pk1, pk2, wv0, wv1, wv2,
            g0, g1, g2, si0, si1, si2, sg0, sg1, sg2, ss0, ss1, ss2):
        cid = lax.axis_index("c")
        sid = lax.axis_index("s")
        r0 = sid * RPT
        nbc = jnp.where(cid == 0, nb0, nb1)
        tb = cid * (NS * nb0) + sid * nbc
        pks = [pk0, pk1, pk2]
        wvs = [wv0, wv1, wv2]
        gs = [g0, g1, g2]
        semI = [si0, si1, si2]
        semG = [sg0, sg1, sg2]
        semS = [ss0, ss1, ss2]
        zf = jnp.zeros((16,), F32)
        zi = jnp.zeros((16,), I32)

        for g in gs:
            for e in range(B):
                for j in range(D // 16):
                    g[e, pl.ds(j * 16, 16)] = zf
        for r in range(2):
            for c in range(B // 16):
                pk2[r, pl.ds(c * 16, 16)] = zi
        for i in range(RPT // B):
            pltpu.sync_copy(g0, acc.at[pl.ds(r0 + i * B, B)])
        plsc.subcore_barrier()

        pltpu.async_copy(g2, acc.at[pk2.at[1]], ss2, add=True)
        pltpu.sync_copy(pk_h.at[tb], pk0)
        pltpu.sync_copy(wt_h.at[tb], wv0)
        pltpu.async_copy(y_h.at[pk0.at[0]], g0, sg0)
        pltpu.async_copy(pk_h.at[tb + 1], pk1, si1)
        pltpu.async_copy(wt_h.at[tb + 1], wv1, si1)

        def phase(b, k):
            m, m1, m2 = k, (k + 1) % 3, (k + 2) % 3
            pltpu.make_async_copy(y_h.at[pks[m].at[0]], gs[m], semG[m]).wait()
            pltpu.make_async_copy(pk_h.at[tb], pks[m1], semI[m1]).wait()
            pltpu.make_async_copy(wt_h.at[tb], wvs[m1], semI[m1]).wait()
            pltpu.async_copy(y_h.at[pks[m1].at[0]], gs[m1], semG[m1])
            for ch in range(B // 16):
                wv = wvs[m][pl.ds(ch * 16, 16)]
                for l in range(16):
                    ws = jnp.full((16,), wv[l], F32)
                    e = ch * 16 + l
                    for j in range(D // 16):
                        sj = pl.ds(j * 16, 16)
                        gs[m][e, sj] = gs[m][e, sj] * ws
            pltpu.async_copy(gs[m], acc.at[pks[m].at[1]], semS[m], add=True)
            pltpu.make_async_copy(gs[m2], acc.at[pks[m2].at[1]], semS[m2]).wait()
            pltpu.async_copy(pk_h.at[tb + b + 2], pks[m2], semI[m2])
            pltpu.async_copy(wt_h.at[tb + b + 2], wvs[m2], semI[m2])

        def triple(gi, c):
            for k in range(3):
                phase(gi * 3 + k, k)
            return c

        lax.fori_loop(0, nbc // 3, triple, 0)
        pltpu.make_async_copy(gs[2], acc.at[pk2.at[1]], ss2).wait()
        pltpu.make_async_copy(y_h.at[pk0.at[0]], g0, sg0).wait()
        pltpu.make_async_copy(pk_h.at[tb], pk1, si1).wait()
        pltpu.make_async_copy(wt_h.at[tb], wv1, si1).wait()
        plsc.subcore_barrier()
        pltpu.sync_copy(acc.at[pl.ds(r0, RPT)], out_h.at[cid, pl.ds(r0, RPT)])

    return a_w


def _prep_body(dg_ref, x_ref, dis_ref, y_ref):
    deg = dg_ref[0] + dg_ref[1]
    dis = jnp.where(deg > 0.0, lax.rsqrt(jnp.maximum(deg, 1e-12)), 0.0)
    dis_ref[...] = dis
    y_ref[...] = dis * x_ref[...]


def _mid_body(z_ref, dis_ref, o_ref):
    z = z_ref[...]
    dis = dis_ref[...]
    o_ref[...] = (dis * dis) * (z[0] + z[1])


def _fin_body(x_ref, dis_ref, z1_ref, z2_ref, w0_ref, w1_ref, w2_ref,
              b_ref, wc_ref, o_ref):
    dis_b = dis_ref[...]
    z1 = z1_ref[...]
    z2 = z2_ref[...]
    a = dis_b * (z1[0] + z1[1])
    bb = dis_b * (z2[0] + z2[1])
    x = x_ref[...]
    dot = functools.partial(jnp.dot, preferred_element_type=F32,
                            precision=lax.Precision.HIGHEST)
    pre = dot(x, w0_ref[...]) + dot(a, w1_ref[...]) + dot(bb, w2_ref[...])
    pre = pre + b_ref[...]
    gi = jax.nn.sigmoid(pre[:, :D])
    gt = jnp.tanh(pre[:, D:2 * D])
    cn = gi * gt
    go = jax.nn.sigmoid(pre[:, 2 * D:] + wc_ref[...] * cn)
    o_ref[...] = go * jnp.tanh(cn)


def kernel(X, edge_weight, Wx, bx, Wh, bh, w_c, b_gate, edge_index):
    E = edge_weight.shape[0]
    row0, col0 = edge_index[0], edge_index[1]

    nb_d = -(-E // (NC * NS * BD))
    e_pad_d = NC * NS * BD * nb_d
    rowf = jnp.concatenate([row0, jnp.zeros((e_pad_d - E,), I32)])
    wf = jnp.concatenate([edge_weight, jnp.zeros((e_pad_d - E,), F32)])

    nb_pt = -(-E // (2 * NS * B))
    nb0 = -(-(2 * nb_pt * 42) // (100 * 3)) * 3
    nb1 = -(-(E - NS * nb0 * B) // (NS * B * 3)) * 3
    nbt = NS * (nb0 + nb1) + 2
    ep = nbt * B
    colp = jnp.concatenate([col0, jnp.zeros((ep - E,), I32)]).reshape(nbt, 1, B)
    rowp = jnp.concatenate([row0, jnp.zeros((ep - E,), I32)]).reshape(nbt, 1, B)
    wt = jnp.concatenate([edge_weight, jnp.zeros((ep - E,), F32)]).reshape(nbt, B)
    pk = jnp.concatenate([colp, rowp], axis=1)

    degp = _deg_kernel(nb_d)(rowf, wf)

    dis2d, y1 = pl.pallas_call(
        _prep_body,
        grid=(N // RB,),
        in_specs=[
            pl.BlockSpec((NC, RB, 1), lambda i: (0, i, 0)),
            pl.BlockSpec((RB, D), lambda i: (i, 0)),
        ],
        out_specs=[
            pl.BlockSpec((RB, 1), lambda i: (i, 0)),
            pl.BlockSpec((RB, D), lambda i: (i, 0)),
        ],
        out_shape=[
            jax.ShapeDtypeStruct((N, 1), F32),
            jax.ShapeDtypeStruct((N, D), F32),
        ],
    )(degp.reshape(NC, NP, 1), X)

    a_w = _a_w_kernel(nb0, nb1)

    z1p = a_w(y1, pk, wt)

    t2 = pl.pallas_call(
        _mid_body,
        grid=(N // RB,),
        in_specs=[
            pl.BlockSpec((NC, RB, D), lambda i: (0, i, 0)),
            pl.BlockSpec((RB, 1), lambda i: (i, 0)),
        ],
        out_specs=pl.BlockSpec((RB, D), lambda i: (i, 0)),
        out_shape=jax.ShapeDtypeStruct((N, D), F32),
    )(z1p, dis2d)

    z2p = a_w(t2, pk, wt)

    g_idx = jnp.array([0, 2, 3])
    Wg = Wx[g_idx]
    w0 = jnp.transpose(Wg[:, 0] - Wg[:, 2], (1, 0, 2)).reshape(D, 3 * D)
    w1 = jnp.transpose(-Wg[:, 1], (1, 0, 2)).reshape(D, 3 * D)
    w2 = jnp.transpose(2.0 * Wg[:, 2], (1, 0, 2)).reshape(D, 3 * D)
    bcat = (bx[g_idx] + bh[g_idx] + b_gate[g_idx, 0]).reshape(1, 3 * D)
    wc2 = w_c[2].reshape(1, D)

    return pl.pallas_call(
        _fin_body,
        grid=(N // RB,),
        in_specs=[
            pl.BlockSpec((RB, D), lambda i: (i, 0)),
            pl.BlockSpec((RB, 1), lambda i: (i, 0)),
            pl.BlockSpec((NC, RB, D), lambda i: (0, i, 0)),
            pl.BlockSpec((NC, RB, D), lambda i: (0, i, 0)),
            pl.BlockSpec((D, 3 * D), lambda i: (0, 0)),
            pl.BlockSpec((D, 3 * D), lambda i: (0, 0)),
            pl.BlockSpec((D, 3 * D), lambda i: (0, 0)),
            pl.BlockSpec((1, 3 * D), lambda i: (0, 0)),
            pl.BlockSpec((1, D), lambda i: (0, 0)),
        ],
        out_specs=pl.BlockSpec((RB, D), lambda i: (i, 0)),
        out_shape=jax.ShapeDtypeStruct((N, D), F32),
    )(X, dis2d, z1p, z2p, w0, w1, w2, bcat, wc2)
